# Initial kernel scaffold; baseline (speedup 1.0000x reference)
import jax
import jax.numpy as jnp
from jax.experimental import pallas as pl


def kernel(x, table, W_down, W_up):
    raise NotImplementedError("write your pallas kernel here")


# speedup vs baseline: 1.1842x; 1.1842x over previous
import functools

import jax
import jax.numpy as jnp
from jax import lax
from jax.experimental import pallas as pl
from jax.experimental.pallas import tpu as pltpu
from jax.experimental.pallas import tpu_sc as plsc

DIM = 64
NC = 2
NS = 16
NW = NC * NS

GATHER_CHUNK = 1024
MM_TILE = 8192


def _sc_gather(table, flat_idx, n):
    per_w = n // NW
    chunk = min(GATHER_CHUNK, per_w)
    n_chunks = per_w // chunk
    mesh = plsc.VectorSubcoreMesh(
        core_axis_name="c", subcore_axis_name="s",
        num_cores=NC, num_subcores=NS)

    @functools.partial(
        pl.kernel,
        out_type=jax.ShapeDtypeStruct((n, DIM), jnp.float32),
        mesh=mesh,
        scratch_types=[
            pltpu.VMEM((chunk,), jnp.int32),
            pltpu.VMEM((chunk, DIM), jnp.float32),
            pltpu.SemaphoreType.DMA,
        ],
        compiler_params=pltpu.CompilerParams(use_tc_tiling_on_sc=False),
    )
    def gather_kernel(table_hbm, idx_hbm, out_hbm, idx_v, rows_v, sem):
        wid = lax.axis_index("s") * NC + lax.axis_index("c")
        base = wid * per_w

        def body(i, carry):
            start = base + i * chunk
            pltpu.sync_copy(idx_hbm.at[pl.ds(start, chunk)], idx_v)
            pltpu.async_copy(table_hbm.at[idx_v], rows_v, sem).wait()
            pltpu.sync_copy(rows_v, out_hbm.at[pl.ds(start, chunk)])
            return carry

        lax.fori_loop(0, n_chunks, body, 0)

    return gather_kernel(table, flat_idx)


def _tc_project(emb, W_down, W_up, n):
    def mm_kernel(emb_ref, wd_ref, wu_ref, out_ref):
        down = jnp.dot(emb_ref[...], wd_ref[...].T,
                       preferred_element_type=jnp.float32)
        out_ref[...] = jnp.dot(down, wu_ref[...].T,
                               preferred_element_type=jnp.float32)

    tile = min(MM_TILE, n)
    return pl.pallas_call(
        mm_kernel,
        grid=(n // tile,),
        in_specs=[
            pl.BlockSpec((tile, DIM), lambda i: (i, 0)),
            pl.BlockSpec((DIM // 2, DIM), lambda i: (0, 0)),
            pl.BlockSpec((DIM, DIM // 2), lambda i: (0, 0)),
        ],
        out_specs=pl.BlockSpec((tile, DIM), lambda i: (i, 0)),
        out_shape=jax.ShapeDtypeStruct((n, DIM), jnp.float32),
    )(emb, W_down, W_up)


@jax.jit
def kernel(x, table, W_down, W_up):
    B, L = x.shape
    n = B * L
    flat_idx = x.reshape(n).astype(jnp.int32)
    emb = _sc_gather(table, flat_idx, n)
    out = _tc_project(emb, W_down, W_up, n)
    return out.reshape(B, L, DIM)


# speedup vs baseline: 2.4046x; 2.0305x over previous
import functools

import jax
import jax.numpy as jnp
from jax import lax
from jax.experimental import pallas as pl
from jax.experimental.pallas import tpu as pltpu
from jax.experimental.pallas import tpu_sc as plsc

DIM = 64
NC = 2
NS = 16
NW = NC * NS

GATHER_CHUNK = 1024
TR_COLS = 8192


def _tc_transform(tableT, W_down, W_up, vocab):
    def tr_kernel(tt_ref, wd_ref, wu_ref, out_ref):
        wc = jnp.dot(wu_ref[...], wd_ref[...],
                     preferred_element_type=jnp.float32)
        z = lax.dot_general(tt_ref[...], wc, (((0,), (1,)), ((), ())),
                            preferred_element_type=jnp.float32)
        out_ref[0, :, 0:DIM] = z[: TR_COLS // 2]
        out_ref[0, :, DIM:128] = z[TR_COLS // 2 :]

    grid = (vocab + TR_COLS - 1) // TR_COLS
    return pl.pallas_call(
        tr_kernel,
        grid=(grid,),
        in_specs=[
            pl.BlockSpec((DIM, TR_COLS), lambda i: (0, i)),
            pl.BlockSpec((DIM // 2, DIM), lambda i: (0, 0)),
            pl.BlockSpec((DIM, DIM // 2), lambda i: (0, 0)),
        ],
        out_specs=pl.BlockSpec((1, TR_COLS // 2, 128), lambda i: (i, 0, 0)),
        out_shape=jax.ShapeDtypeStruct((grid, TR_COLS // 2, 128), jnp.float32),
    )(tableT, W_down, W_up)


def _sc_gather(src, flat_idx, n):
    per_w = n // NW
    chunk = min(GATHER_CHUNK, per_w)
    n_chunks = per_w // chunk
    mesh = plsc.VectorSubcoreMesh(
        core_axis_name="c", subcore_axis_name="s",
        num_cores=NC, num_subcores=NS)

    @functools.partial(
        pl.kernel,
        out_type=jax.ShapeDtypeStruct((n, DIM), jnp.float32),
        mesh=mesh,
        scratch_types=[
            pltpu.VMEM((chunk,), jnp.int32),
            pltpu.VMEM((chunk, DIM), jnp.float32),
            pltpu.SemaphoreType.DMA,
        ],
        compiler_params=pltpu.CompilerParams(use_tc_tiling_on_sc=False),
    )
    def gather_kernel(src_hbm, idx_hbm, out_hbm, idx_v, rows_v, sem):
        wid = lax.axis_index("s") * NC + lax.axis_index("c")
        base = wid * per_w

        def body(i, carry):
            start = base + i * chunk
            pltpu.sync_copy(idx_hbm.at[pl.ds(start, chunk)], idx_v)
            pltpu.async_copy(src_hbm.at[idx_v], rows_v, sem).wait()
            pltpu.sync_copy(rows_v, out_hbm.at[pl.ds(start, chunk)])
            return carry

        lax.fori_loop(0, n_chunks, body, 0)

    return gather_kernel(src, flat_idx)


@jax.jit
def kernel(x, table, W_down, W_up):
    B, L = x.shape
    vocab = table.shape[0]
    n = B * L
    flat_idx = x.reshape(n).astype(jnp.int32)
    blk = flat_idx >> 13
    off = flat_idx & 8191
    phys_idx = (blk << 13) + ((off & 4095) << 1) + (off >> 12)
    g = _tc_transform(table.T, W_down, W_up, vocab)
    g_rows = g.reshape(g.shape[0] * g.shape[1] * 2, DIM)
    out = _sc_gather(g_rows, phys_idx, n)
    return out.reshape(B, L, DIM)
